# baseline (speedup 1.0000x reference)
import functools

import jax
import jax.numpy as jnp
from jax import lax
from jax.experimental import pallas as pl
from jax.experimental.pallas import tpu as pltpu
from jax.experimental.pallas import tpu_sc as plsc

N = 10000
E = 160000
F = 256
HW = 128
NCORE = 2
NSUB = 16
EPB = E // NSUB
CH = 128
KCH = 79
EPAD = KCH * CH
NPAD = 10240
RPB = NPAD // NSUB
APB = NPAD // NSUB
WR = 128
KB = 40
_BATCHES = ((0, 40), (40, 39))
EPS = 1e-8

_mesh = plsc.VectorSubcoreMesh(
    core_axis_name="c", subcore_axis_name="s", num_cores=NCORE, num_subcores=NSUB
)


def _newton_rsqrt(p):
    i = plsc.bitcast(p, jnp.int32)
    y = plsc.bitcast(jnp.int32(0x5F3759DF) - (i >> 1), jnp.float32)
    for _ in range(3):
        y = y * (1.5 - 0.5 * p * y * y)
    return y


def _stage_edges(src3, tgt3, s, src_v, tgt_v):
    pltpu.sync_copy(tgt3.at[s], tgt_v)
    pltpu.sync_copy(src3.at[s], src_v)


def _zero_acc(rows_v, acc_sh, s):
    for i in range(5):
        pltpu.sync_copy(rows_v, acc_sh.at[pl.ds(s * APB + i * WR, WR), :])


def _build_zeros(rows_v):
    z16 = jnp.zeros((16,), jnp.float32)

    def zb(i, carry):
        for j in range(HW // 16):
            rows_v[i, pl.ds(j * 16, 16)] = z16
        return carry

    lax.fori_loop(0, WR, zb, 0)


def _adjust_src(src_v, c):
    def adj(k, carry):
        for g in range(CH // 16):
            src_v[k, pl.ds(g * 16, 16)] = src_v[k, pl.ds(g * 16, 16)] + c * N
        return carry

    lax.fori_loop(0, KCH, adj, 0)


def _scale_rows(rows_v, norm_v, g):
    def sc(jg, c2):
        n16 = norm_v[g, pl.ds(jg * 16, 16)]
        for l in range(16):
            nj = n16[l]
            j = jg * 16 + l
            for i in range(HW // 16):
                rows_v[j, pl.ds(i * 16, 16)] = rows_v[j, pl.ds(i * 16, 16)] * nj
        return c2

    lax.fori_loop(0, CH // 16, sc, 0, unroll=2)


def _ring(cnt, xcat, src_v, tgt_v, norm_v, bufs, acc_sh):
    def fire_gather(g, b):
        buf, semg, _ = bufs[b]
        pltpu.async_copy(xcat.at[src_v.at[g]], buf, semg)

    def step(g, b, prefetch):
        buf, semg, sems = bufs[b]
        pltpu.make_async_copy(xcat.at[src_v.at[g]], buf, semg).wait()
        _scale_rows(buf, norm_v, g)
        pltpu.async_copy(buf, acc_sh.at[tgt_v.at[g]], sems, add=True)
        if prefetch:
            pltpu.make_async_copy(buf, acc_sh.at[tgt_v.at[g]], sems).wait()
            fire_gather(g + 2, b)

    fire_gather(0, 0)
    fire_gather(1, 1)
    npairs = (cnt - 2) // 2 if cnt % 2 == 0 else (cnt - 3) // 2

    def body(i, carry):
        g = i * 2
        step(g, 0, True)
        step(g + 1, 1, True)
        return carry

    lax.fori_loop(0, npairs, body, 0)
    if cnt % 2 == 0:
        tail = [(cnt - 2, 0), (cnt - 1, 1)]
    else:
        step(cnt - 3, 0, True)
        tail = [(cnt - 2, 1), (cnt - 1, 0)]
    for g, b in tail:
        step(g, b, False)
    for b in range(2):
        buf, _, sems = bufs[b]
        pltpu.make_async_copy(buf, acc_sh.at[tgt_v.at[0]], sems).wait()


def _aggregate(xcat, src3, tgt3, norm3, c, s, src_v, tgt_v, norm_v,
               ra_v, rb_v, acc_sh, sem_ga, sem_gb, sem_sa, sem_sb):
    bufs = ((ra_v, sem_ga, sem_sa), (rb_v, sem_gb, sem_sb))
    for off, cnt in _BATCHES:
        pltpu.sync_copy(src3.at[s, pl.ds(off, cnt)], src_v.at[pl.ds(0, cnt)])
        pltpu.sync_copy(tgt3.at[s, pl.ds(off, cnt)], tgt_v.at[pl.ds(0, cnt)])
        pltpu.sync_copy(norm3.at[s, pl.ds(off, cnt)], norm_v.at[pl.ds(0, cnt)])
        _adjust_src_b(src_v, c, cnt)
        _ring(cnt, xcat, src_v, tgt_v, norm_v, bufs, acc_sh)


def _adjust_src_b(src_v, c, cnt):
    def adj(k, carry):
        for g in range(CH // 16):
            src_v[k, pl.ds(g * 16, 16)] = src_v[k, pl.ds(g * 16, 16)] + c * N
        return carry

    lax.fori_loop(0, cnt, adj, 0)


def _writeout(acc_sh, agg, c, s):
    for i in range(5):
        r0 = s * APB + i * WR
        pltpu.sync_copy(acc_sh.at[pl.ds(r0, WR), :], agg.at[pl.ds(c * NPAD + r0, WR), :])


def _gcn_deg(src3, tgt3, degout, norm3,
             dega_sh,
             zer16_v, ones_v, src_v, tgt_v, norm_v, deg_v, sem_d):
    c = lax.axis_index("c")
    s = lax.axis_index("s")
    z16 = jnp.zeros((16,), jnp.float32)

    def zb16(i, carry):
        zer16_v[pl.ds(i * 16, 16)] = z16
        return carry

    lax.fori_loop(0, RPB // 16, zb16, 0)
    one16 = jnp.full((16,), 1.0, jnp.float32)

    def ob(i, carry):
        ones_v[pl.ds(i * 16, 16)] = one16
        return carry

    lax.fori_loop(0, CH // 16, ob, 0)

    pltpu.sync_copy(zer16_v, dega_sh.at[pl.ds(s * RPB, RPB)])
    _stage_edges(src3, tgt3, s, src_v, tgt_v)
    plsc.subcore_barrier()

    def dfire(k, carry):
        pltpu.async_copy(ones_v, dega_sh.at[tgt_v.at[k]], sem_d, add=True)
        return carry

    lax.fori_loop(0, KCH, dfire, 0)

    def ddrain(k, carry):
        pltpu.make_async_copy(ones_v, dega_sh.at[tgt_v.at[k]], sem_d).wait()
        return carry

    lax.fori_loop(0, KCH, ddrain, 0)
    plsc.subcore_barrier()

    pltpu.sync_copy(dega_sh, deg_v)

    @pl.when(c == 0)
    def _():
        pltpu.sync_copy(
            dega_sh.at[pl.ds(s * RPB, RPB)], degout.at[pl.ds(s * RPB, RPB)]
        )

    def nrm(k, carry):
        for g in range(CH // 16):
            t16 = tgt_v[k, pl.ds(g * 16, 16)]
            s16 = src_v[k, pl.ds(g * 16, 16)]
            dt = plsc.load_gather(deg_v, [t16])
            dsr = plsc.load_gather(deg_v, [s16])
            prod = dt * dsr
            y = _newton_rsqrt(prod)
            norm_v[k, pl.ds(g * 16, 16)] = jnp.where(
                prod == 0.0, jnp.float32(1e8), y
            )
        return carry

    lax.fori_loop(0, KCH, nrm, 0)

    @pl.when(c == 0)
    def _():
        pltpu.sync_copy(norm_v, norm3.at[s])


def _gcn_k2(xcat, src3, tgt3, norm3, agg,
            acc_sh, src_v, tgt_v, norm_v, ra_v, rb_v,
            sem_ga, sem_gb, sem_sa, sem_sb):
    c = lax.axis_index("c")
    s = lax.axis_index("s")

    _build_zeros(ra_v)
    _zero_acc(ra_v, acc_sh, s)
    plsc.subcore_barrier()

    _aggregate(xcat, src3, tgt3, norm3, c, s, src_v, tgt_v, norm_v,
               ra_v, rb_v, acc_sh, sem_ga, sem_gb, sem_sa, sem_sb)
    plsc.subcore_barrier()
    _writeout(acc_sh, agg, c, s)


_SC_PARAMS = pltpu.CompilerParams(needs_layout_passes=False)

_scdeg = pl.kernel(
    _gcn_deg,
    compiler_params=_SC_PARAMS,
    out_type=[
        jax.ShapeDtypeStruct((NPAD,), jnp.float32),
        jax.ShapeDtypeStruct((NSUB, KCH, CH), jnp.float32),
    ],
    mesh=_mesh,
    scratch_types=[
        pltpu.VMEM_SHARED((NPAD,), jnp.float32),
        pltpu.VMEM((RPB,), jnp.float32),
        pltpu.VMEM((CH,), jnp.float32),
        pltpu.VMEM((KCH, CH), jnp.int32),
        pltpu.VMEM((KCH, CH), jnp.int32),
        pltpu.VMEM((KCH, CH), jnp.float32),
        pltpu.VMEM((NPAD,), jnp.float32),
        pltpu.SemaphoreType.DMA,
    ],
)

_scagg = pl.kernel(
    _gcn_k2,
    compiler_params=_SC_PARAMS,
    out_type=[jax.ShapeDtypeStruct((2 * NPAD, HW), jnp.float32)],
    mesh=_mesh,
    scratch_types=[
        pltpu.VMEM_SHARED((NPAD, HW), jnp.float32),
        pltpu.VMEM((KB, CH), jnp.int32),
        pltpu.VMEM((KB, CH), jnp.int32),
        pltpu.VMEM((KB, CH), jnp.float32),
        pltpu.VMEM((CH, HW), jnp.float32),
        pltpu.VMEM((CH, HW), jnp.float32),
        pltpu.SemaphoreType.DMA,
        pltpu.SemaphoreType.DMA,
        pltpu.SemaphoreType.DMA,
        pltpu.SemaphoreType.DMA,
    ],
)


_RB = 2000


def _tc1_body(a_ref, deg_ref, w1_ref, b1_ref, w2_ref, o_ref):
    a = a_ref[...]
    h = lax.dot_general(a, w1_ref[...], (((1,), (1,)), ((), ())),
                        preferred_element_type=jnp.float32)
    h = h + deg_ref[...] * b1_ref[...][None, :]
    h = jnp.where(h >= 0, h, 0.01 * h)
    o_ref[...] = lax.dot_general(h, w2_ref[...], (((1,), (1,)), ((), ())),
                                 preferred_element_type=jnp.float32)


def _tc2_body(a_ref, deg_ref, b2_ref, w3_ref, b3_ref, o_ref):
    a = a_ref[...] + deg_ref[...] * b2_ref[...][None, :]
    a = jnp.where(a >= 0, a, 0.01 * a)
    l = lax.dot_general(a, w3_ref[...], (((1,), (1,)), ((), ())),
                        preferred_element_type=jnp.float32)
    l = l + b3_ref[...][None, :] + EPS
    m = jnp.max(l, axis=1, keepdims=True)
    e = jnp.exp(l - m)
    o_ref[...] = l - m - jnp.log(jnp.sum(e, axis=1, keepdims=True))


def _tc1(a1, deg, W1, b1, W2):
    grid = (N // _RB,)
    return pl.pallas_call(
        _tc1_body,
        grid=grid,
        in_specs=[
            pl.BlockSpec((_RB, F), lambda i: (i, 0)),
            pl.BlockSpec((_RB, 1), lambda i: (i, 0)),
            pl.BlockSpec((512, F), lambda i: (0, 0)),
            pl.BlockSpec((512,), lambda i: (0,)),
            pl.BlockSpec((F, 512), lambda i: (0, 0)),
        ],
        out_specs=pl.BlockSpec((_RB, F), lambda i: (i, 0)),
        out_shape=jax.ShapeDtypeStruct((N, F), jnp.float32),
    )(a1, deg, W1, b1, W2)


def _tc2(a2, deg, b2, W3, b3):
    grid = (N // _RB,)
    return pl.pallas_call(
        _tc2_body,
        grid=grid,
        in_specs=[
            pl.BlockSpec((_RB, F), lambda i: (i, 0)),
            pl.BlockSpec((_RB, 1), lambda i: (i, 0)),
            pl.BlockSpec((F,), lambda i: (0,)),
            pl.BlockSpec((64, F), lambda i: (0, 0)),
            pl.BlockSpec((64,), lambda i: (0,)),
        ],
        out_specs=pl.BlockSpec((_RB, 64), lambda i: (i, 0)),
        out_shape=jax.ShapeDtypeStruct((N, 64), jnp.float32),
    )(a2, deg, b2, W3, b3)


def kernel(node_feature, edge_index, graph_index, supernode_index,
           W1, b1, W2, b2, W3, b3):
    pad = ((0, 0), (0, EPAD - EPB))
    src3 = jnp.pad(edge_index[0].reshape(NSUB, EPB), pad).reshape(NSUB, KCH, CH)
    tgt3 = jnp.pad(
        edge_index[1].reshape(NSUB, EPB), pad, constant_values=N
    ).reshape(NSUB, KCH, CH)
    xcat = jnp.concatenate([node_feature[:, :HW], node_feature[:, HW:]], axis=0)

    degp, norm3 = _scdeg(src3, tgt3)
    deg = degp[:N]
    (agg1c,) = _scagg(xcat, src3, tgt3, norm3)
    a1 = jnp.concatenate([agg1c[:N], agg1c[NPAD:NPAD + N]], axis=1)
    p = _tc1(a1, deg[:, None], W1, b1, W2)

    pcat = jnp.concatenate([p[:, :HW], p[:, HW:]], axis=0)
    (agg2c,) = _scagg(pcat, src3, tgt3, norm3)
    a2 = jnp.concatenate([agg2c[:N], agg2c[NPAD:NPAD + N]], axis=1)
    return _tc2(a2, deg[:, None], b2, W3, b3)


# speedup vs baseline: 1.0430x; 1.0430x over previous
import functools

import jax
import jax.numpy as jnp
from jax import lax
from jax.experimental import pallas as pl
from jax.experimental.pallas import tpu as pltpu
from jax.experimental.pallas import tpu_sc as plsc

N = 10000
E = 160000
F = 256
HW = 128
NCORE = 2
NSUB = 16
EPB = E // NSUB
CH = 128
KCH = 79
EPAD = KCH * CH
NPAD = 10240
RPB = NPAD // NSUB
APB = NPAD // NSUB
WR = 128
KB = 40
_BATCHES = ((0, 40), (40, 39))
EPS = 1e-8

_mesh = plsc.VectorSubcoreMesh(
    core_axis_name="c", subcore_axis_name="s", num_cores=NCORE, num_subcores=NSUB
)


def _newton_rsqrt(p):
    i = plsc.bitcast(p, jnp.int32)
    y = plsc.bitcast(jnp.int32(0x5F3759DF) - (i >> 1), jnp.float32)
    for _ in range(3):
        y = y * (1.5 - 0.5 * p * y * y)
    return y


def _stage_edges(src3, tgt3, s, src_v, tgt_v):
    pltpu.sync_copy(tgt3.at[s], tgt_v)
    pltpu.sync_copy(src3.at[s], src_v)


def _zero_acc(rows_v, acc_sh, s):
    for i in range(5):
        pltpu.sync_copy(rows_v, acc_sh.at[pl.ds(s * APB + i * WR, WR), :])


def _build_zeros(rows_v):
    z16 = jnp.zeros((16,), jnp.float32)

    def zb(i, carry):
        for j in range(HW // 16):
            rows_v[i, pl.ds(j * 16, 16)] = z16
        return carry

    lax.fori_loop(0, WR, zb, 0)


def _adjust_src(src_v, c):
    def adj(k, carry):
        for g in range(CH // 16):
            src_v[k, pl.ds(g * 16, 16)] = src_v[k, pl.ds(g * 16, 16)] + c * N
        return carry

    lax.fori_loop(0, KCH, adj, 0)


def _scale_rows(rows_v, norm_v, g):
    def sc(jg, c2):
        n16 = norm_v[g, pl.ds(jg * 16, 16)]
        for l in range(16):
            nj = n16[l]
            j = jg * 16 + l
            for i in range(HW // 16):
                rows_v[j, pl.ds(i * 16, 16)] = rows_v[j, pl.ds(i * 16, 16)] * nj
        return c2

    lax.fori_loop(0, CH // 16, sc, 0, unroll=2)


def _ring(cnt, xcat, src_v, tgt_v, norm_v, bufs, acc_sh):
    def fire_gather(g, b):
        buf, semg, _ = bufs[b]
        pltpu.async_copy(xcat.at[src_v.at[g]], buf, semg)

    def step(g, b, prefetch):
        buf, semg, sems = bufs[b]
        pltpu.make_async_copy(xcat.at[src_v.at[g]], buf, semg).wait()
        _scale_rows(buf, norm_v, g)
        pltpu.async_copy(buf, acc_sh.at[tgt_v.at[g]], sems, add=True)
        if prefetch:
            pltpu.make_async_copy(buf, acc_sh.at[tgt_v.at[g]], sems).wait()
            fire_gather(g + 2, b)

    fire_gather(0, 0)
    fire_gather(1, 1)
    npairs = (cnt - 2) // 2 if cnt % 2 == 0 else (cnt - 3) // 2

    def body(i, carry):
        g = i * 2
        step(g, 0, True)
        step(g + 1, 1, True)
        return carry

    lax.fori_loop(0, npairs, body, 0)
    if cnt % 2 == 0:
        tail = [(cnt - 2, 0), (cnt - 1, 1)]
    else:
        step(cnt - 3, 0, True)
        tail = [(cnt - 2, 1), (cnt - 1, 0)]
    for g, b in tail:
        step(g, b, False)
    for b in range(2):
        buf, _, sems = bufs[b]
        pltpu.make_async_copy(buf, acc_sh.at[tgt_v.at[0]], sems).wait()


def _aggregate(xcat, src3, tgt3, norm3, c, s, src_v, tgt_v, norm_v,
               ra_v, rb_v, acc_sh, sem_ga, sem_gb, sem_sa, sem_sb):
    bufs = ((ra_v, sem_ga, sem_sa), (rb_v, sem_gb, sem_sb))
    for off, cnt in _BATCHES:
        pltpu.sync_copy(src3.at[s, pl.ds(off, cnt)], src_v.at[pl.ds(0, cnt)])
        pltpu.sync_copy(tgt3.at[s, pl.ds(off, cnt)], tgt_v.at[pl.ds(0, cnt)])
        pltpu.sync_copy(norm3.at[s, pl.ds(off, cnt)], norm_v.at[pl.ds(0, cnt)])
        _adjust_src_b(src_v, c, cnt)
        _ring(cnt, xcat, src_v, tgt_v, norm_v, bufs, acc_sh)


def _adjust_src_b(src_v, c, cnt):
    def adj(k, carry):
        for g in range(CH // 16):
            src_v[k, pl.ds(g * 16, 16)] = src_v[k, pl.ds(g * 16, 16)] * 2 + c
        return carry

    lax.fori_loop(0, cnt, adj, 0)


def _writeout(acc_sh, agg, c, s):
    for i in range(5):
        r0 = s * APB + i * WR
        pltpu.sync_copy(acc_sh.at[pl.ds(r0, WR), :], agg.at[pl.ds(c * NPAD + r0, WR), :])


def _gcn_deg(src3, tgt3, degout, norm3,
             dega_sh,
             zer16_v, ones_v, src_v, tgt_v, norm_v, deg_v, sem_d):
    c = lax.axis_index("c")
    s = lax.axis_index("s")
    z16 = jnp.zeros((16,), jnp.float32)

    def zb16(i, carry):
        zer16_v[pl.ds(i * 16, 16)] = z16
        return carry

    lax.fori_loop(0, RPB // 16, zb16, 0)
    one16 = jnp.full((16,), 1.0, jnp.float32)

    def ob(i, carry):
        ones_v[pl.ds(i * 16, 16)] = one16
        return carry

    lax.fori_loop(0, CH // 16, ob, 0)

    pltpu.sync_copy(zer16_v, dega_sh.at[pl.ds(s * RPB, RPB)])
    _stage_edges(src3, tgt3, s, src_v, tgt_v)
    plsc.subcore_barrier()

    def dfire(k, carry):
        pltpu.async_copy(ones_v, dega_sh.at[tgt_v.at[k]], sem_d, add=True)
        return carry

    lax.fori_loop(0, KCH, dfire, 0)

    def ddrain(k, carry):
        pltpu.make_async_copy(ones_v, dega_sh.at[tgt_v.at[k]], sem_d).wait()
        return carry

    lax.fori_loop(0, KCH, ddrain, 0)
    plsc.subcore_barrier()

    pltpu.sync_copy(dega_sh, deg_v)

    @pl.when(c == 0)
    def _():
        pltpu.sync_copy(
            dega_sh.at[pl.ds(s * RPB, RPB)], degout.at[pl.ds(s * RPB, RPB)]
        )

    def nrm(k, carry):
        for g in range(CH // 16):
            t16 = tgt_v[k, pl.ds(g * 16, 16)]
            s16 = src_v[k, pl.ds(g * 16, 16)]
            dt = plsc.load_gather(deg_v, [t16])
            dsr = plsc.load_gather(deg_v, [s16])
            prod = dt * dsr
            y = _newton_rsqrt(prod)
            norm_v[k, pl.ds(g * 16, 16)] = jnp.where(
                prod == 0.0, jnp.float32(1e8), y
            )
        return carry

    lax.fori_loop(0, KCH, nrm, 0)

    @pl.when(c == 0)
    def _():
        pltpu.sync_copy(norm_v, norm3.at[s])


def _gcn_k2(xcat, src3, tgt3, norm3, agg,
            acc_sh, src_v, tgt_v, norm_v, ra_v, rb_v,
            sem_ga, sem_gb, sem_sa, sem_sb):
    c = lax.axis_index("c")
    s = lax.axis_index("s")

    _build_zeros(ra_v)
    _zero_acc(ra_v, acc_sh, s)
    plsc.subcore_barrier()

    _aggregate(xcat, src3, tgt3, norm3, c, s, src_v, tgt_v, norm_v,
               ra_v, rb_v, acc_sh, sem_ga, sem_gb, sem_sa, sem_sb)
    plsc.subcore_barrier()
    _writeout(acc_sh, agg, c, s)


_SC_PARAMS = pltpu.CompilerParams(needs_layout_passes=False)

_scdeg = pl.kernel(
    _gcn_deg,
    compiler_params=_SC_PARAMS,
    out_type=[
        jax.ShapeDtypeStruct((NPAD,), jnp.float32),
        jax.ShapeDtypeStruct((NSUB, KCH, CH), jnp.float32),
    ],
    mesh=_mesh,
    scratch_types=[
        pltpu.VMEM_SHARED((NPAD,), jnp.float32),
        pltpu.VMEM((RPB,), jnp.float32),
        pltpu.VMEM((CH,), jnp.float32),
        pltpu.VMEM((KCH, CH), jnp.int32),
        pltpu.VMEM((KCH, CH), jnp.int32),
        pltpu.VMEM((KCH, CH), jnp.float32),
        pltpu.VMEM((NPAD,), jnp.float32),
        pltpu.SemaphoreType.DMA,
    ],
)

_scagg = pl.kernel(
    _gcn_k2,
    compiler_params=_SC_PARAMS,
    out_type=[jax.ShapeDtypeStruct((2 * NPAD, HW), jnp.float32)],
    mesh=_mesh,
    scratch_types=[
        pltpu.VMEM_SHARED((NPAD, HW), jnp.float32),
        pltpu.VMEM((KB, CH), jnp.int32),
        pltpu.VMEM((KB, CH), jnp.int32),
        pltpu.VMEM((KB, CH), jnp.float32),
        pltpu.VMEM((CH, HW), jnp.float32),
        pltpu.VMEM((CH, HW), jnp.float32),
        pltpu.SemaphoreType.DMA,
        pltpu.SemaphoreType.DMA,
        pltpu.SemaphoreType.DMA,
        pltpu.SemaphoreType.DMA,
    ],
)


_RB = 1280
_GRID = NPAD // _RB


def _tc1_body(alo_ref, ahi_ref, deg_ref, w1_ref, b1_ref, w2_ref, o_ref):
    h = lax.dot_general(alo_ref[...], w1_ref[:, :HW], (((1,), (1,)), ((), ())),
                        preferred_element_type=jnp.float32)
    h = h + lax.dot_general(ahi_ref[...], w1_ref[:, HW:], (((1,), (1,)), ((), ())),
                            preferred_element_type=jnp.float32)
    h = h + deg_ref[...] * b1_ref[...][None, :]
    h = jnp.where(h >= 0, h, 0.01 * h)
    p_lo = lax.dot_general(h, w2_ref[:HW, :], (((1,), (1,)), ((), ())),
                           preferred_element_type=jnp.float32)
    p_hi = lax.dot_general(h, w2_ref[HW:, :], (((1,), (1,)), ((), ())),
                           preferred_element_type=jnp.float32)
    o_ref[...] = jnp.concatenate([p_lo[:, None, :], p_hi[:, None, :]], axis=1)


def _tc2_body(alo_ref, ahi_ref, deg_ref, b2_ref, w3_ref, b3_ref, o_ref):
    deg = deg_ref[...]
    a_lo = alo_ref[...] + deg * b2_ref[...][None, :HW]
    a_hi = ahi_ref[...] + deg * b2_ref[...][None, HW:]
    a_lo = jnp.where(a_lo >= 0, a_lo, 0.01 * a_lo)
    a_hi = jnp.where(a_hi >= 0, a_hi, 0.01 * a_hi)
    l = lax.dot_general(a_lo, w3_ref[:, :HW], (((1,), (1,)), ((), ())),
                        preferred_element_type=jnp.float32)
    l = l + lax.dot_general(a_hi, w3_ref[:, HW:], (((1,), (1,)), ((), ())),
                            preferred_element_type=jnp.float32)
    l = l + b3_ref[...][None, :] + EPS
    m = jnp.max(l, axis=1, keepdims=True)
    e = jnp.exp(l - m)
    o_ref[...] = l - m - jnp.log(jnp.sum(e, axis=1, keepdims=True))


def _tc1(aggc, deg, W1, b1, W2):
    return pl.pallas_call(
        _tc1_body,
        grid=(_GRID,),
        in_specs=[
            pl.BlockSpec((_RB, HW), lambda i: (i, 0)),
            pl.BlockSpec((_RB, HW), lambda i: (i + _GRID, 0)),
            pl.BlockSpec((_RB, 1), lambda i: (i, 0)),
            pl.BlockSpec((512, F), lambda i: (0, 0)),
            pl.BlockSpec((512,), lambda i: (0,)),
            pl.BlockSpec((F, 512), lambda i: (0, 0)),
        ],
        out_specs=pl.BlockSpec((_RB, 2, HW), lambda i: (i, 0, 0)),
        out_shape=jax.ShapeDtypeStruct((NPAD, 2, HW), jnp.float32),
    )(aggc, aggc, deg, W1, b1, W2)


def _tc2(aggc, deg, b2, W3, b3):
    return pl.pallas_call(
        _tc2_body,
        grid=(_GRID,),
        in_specs=[
            pl.BlockSpec((_RB, HW), lambda i: (i, 0)),
            pl.BlockSpec((_RB, HW), lambda i: (i + _GRID, 0)),
            pl.BlockSpec((_RB, 1), lambda i: (i, 0)),
            pl.BlockSpec((F,), lambda i: (0,)),
            pl.BlockSpec((64, F), lambda i: (0, 0)),
            pl.BlockSpec((64,), lambda i: (0,)),
        ],
        out_specs=pl.BlockSpec((_RB, 64), lambda i: (i, 0)),
        out_shape=jax.ShapeDtypeStruct((NPAD, 64), jnp.float32),
    )(aggc, aggc, deg, b2, W3, b3)


def kernel(node_feature, edge_index, graph_index, supernode_index,
           W1, b1, W2, b2, W3, b3):
    pad = ((0, 0), (0, EPAD - EPB))
    src3 = jnp.pad(edge_index[0].reshape(NSUB, EPB), pad).reshape(NSUB, KCH, CH)
    tgt3 = jnp.pad(
        edge_index[1].reshape(NSUB, EPB), pad, constant_values=N
    ).reshape(NSUB, KCH, CH)

    degp, norm3 = _scdeg(src3, tgt3)
    deg2 = degp[:, None]

    x2 = node_feature.reshape(2 * N, HW)
    (agg1c,) = _scagg(x2, src3, tgt3, norm3)
    p3 = _tc1(agg1c, deg2, W1, b1, W2)
    p2 = p3.reshape(2 * NPAD, HW)
    (agg2c,) = _scagg(p2, src3, tgt3, norm3)
    return _tc2(agg2c, deg2, b2, W3, b3)[:N]
